# Initial kernel scaffold; baseline (speedup 1.0000x reference)
import jax
import jax.numpy as jnp
from jax.experimental import pallas as pl


def kernel(embedded_mesh_features, embedded_mesh2mesh_edge_features, mesh2mesh_edge_indices_src, mesh2mesh_edge_indices_dst, edge_w1, edge_b1, edge_w2, edge_b2, edge_ln_scale, edge_ln_bias, node_w1, node_b1, node_w2, node_b2, node_ln_scale, node_ln_bias):
    raise NotImplementedError("write your pallas kernel here")


# speedup vs baseline: 2.7053x; 2.7053x over previous
import functools

import jax
import jax.numpy as jnp
from jax import lax
from jax.experimental import pallas as pl
from jax.experimental.pallas import tpu as pltpu
from jax.experimental.pallas import tpu_sc as plsc

_NC = 2
_NS = 16
_NW = _NC * _NS


def _chunk_size(per_worker):
    for ch in range(128, 0, -8):
        if per_worker % ch == 0:
            return ch
    raise ValueError(per_worker)


def _sc_mesh():
    return plsc.VectorSubcoreMesh(
        core_axis_name="c", subcore_axis_name="s",
        num_cores=_NC, num_subcores=_NS)


def _gather_sc_body(EW, CH, NCH,
                    n_hbm, src_hbm, dst_hbm, out_s, out_d,
                    idx_s, idx_d, rows_s, rows_d, sem_s, sem_d):
    wid = lax.axis_index("s") * _NC + lax.axis_index("c")
    base = wid * EW

    def body(j, carry):
        row0 = base + j * CH
        pltpu.sync_copy(src_hbm.at[pl.ds(row0, CH)], idx_s)
        pltpu.sync_copy(dst_hbm.at[pl.ds(row0, CH)], idx_d)
        cp_s = pltpu.async_copy(n_hbm.at[idx_s], rows_s, sem_s)
        cp_d = pltpu.async_copy(n_hbm.at[idx_d], rows_d, sem_d)
        cp_s.wait()
        pltpu.sync_copy(rows_s, out_s.at[pl.ds(row0, CH)])
        cp_d.wait()
        pltpu.sync_copy(rows_d, out_d.at[pl.ds(row0, CH)])
        return carry

    lax.fori_loop(0, NCH, body, 0)


def _scatter_sc_body(EW, CH, NCH,
                     e_hbm, dst_hbm, zero_hbm, agg_hbm,
                     idx_v, rows_v, acc, sem):
    cid = lax.axis_index("c")
    sid = lax.axis_index("s")
    wid = sid * _NC + cid

    @pl.when(sid == 0)
    def _():
        pltpu.sync_copy(zero_hbm, acc)

    plsc.subcore_barrier()

    def body(j, carry):
        row0 = wid * EW + j * CH
        pltpu.sync_copy(dst_hbm.at[pl.ds(row0, CH)], idx_v)
        pltpu.sync_copy(e_hbm.at[pl.ds(row0, CH)], rows_v)
        pltpu.sync_copy(rows_v, acc.at[idx_v], add=True)
        return carry

    lax.fori_loop(0, NCH, body, 0)
    plsc.subcore_barrier()

    @pl.when(sid == 0)
    def _():
        pltpu.sync_copy(acc, agg_hbm.at[cid])


@functools.cache
def _make_gather(N, E, D):
    EW = E // _NW
    CH = _chunk_size(EW)
    NCH = EW // CH
    return pl.kernel(
        functools.partial(_gather_sc_body, EW, CH, NCH),
        out_type=(jax.ShapeDtypeStruct((E, D), jnp.float32),
                  jax.ShapeDtypeStruct((E, D), jnp.float32)),
        mesh=_sc_mesh(),
        scratch_types=[
            pltpu.VMEM((CH,), jnp.int32),
            pltpu.VMEM((CH,), jnp.int32),
            pltpu.VMEM((CH, D), jnp.float32),
            pltpu.VMEM((CH, D), jnp.float32),
            pltpu.SemaphoreType.DMA,
            pltpu.SemaphoreType.DMA,
        ])


@functools.cache
def _make_scatter(N, E, D):
    EW = E // _NW
    CH = _chunk_size(EW)
    NCH = EW // CH
    return pl.kernel(
        functools.partial(_scatter_sc_body, EW, CH, NCH),
        out_type=jax.ShapeDtypeStruct((_NC, N, D), jnp.float32),
        mesh=_sc_mesh(),
        scratch_types=[
            pltpu.VMEM((CH,), jnp.int32),
            pltpu.VMEM((CH, D), jnp.float32),
            pltpu.VMEM_SHARED((N, D), jnp.float32),
            pltpu.SemaphoreType.DMA,
        ])


def _edge_mlp_body(D, e_ref, ns_ref, nd_ref, w1_ref, b1_ref, w2_ref, b2_ref,
                   ls_ref, lb_ref, o_ref):
    x = e_ref[...]
    h = jnp.dot(x, w1_ref[0:D], preferred_element_type=jnp.float32)
    h = h + jnp.dot(ns_ref[...], w1_ref[D:2 * D],
                    preferred_element_type=jnp.float32)
    h = h + jnp.dot(nd_ref[...], w1_ref[2 * D:3 * D],
                    preferred_element_type=jnp.float32)
    h = h + b1_ref[...]
    h = h * jax.nn.sigmoid(h)
    h = jnp.dot(h, w2_ref[...], preferred_element_type=jnp.float32)
    h = h + b2_ref[...]
    mu = jnp.mean(h, axis=-1, keepdims=True)
    d = h - mu
    var = jnp.mean(d * d, axis=-1, keepdims=True)
    o_ref[...] = x + ls_ref[...] * d * lax.rsqrt(var + 1e-5) + lb_ref[...]


def _node_mlp_body(D, n_ref, a0_ref, a1_ref, w1_ref, b1_ref, w2_ref, b2_ref,
                   ls_ref, lb_ref, o_ref):
    x = n_ref[...]
    a = a0_ref[...] + a1_ref[...]
    h = jnp.dot(x, w1_ref[0:D], preferred_element_type=jnp.float32)
    h = h + jnp.dot(a, w1_ref[D:2 * D], preferred_element_type=jnp.float32)
    h = h + b1_ref[...]
    h = h * jax.nn.sigmoid(h)
    h = jnp.dot(h, w2_ref[...], preferred_element_type=jnp.float32)
    h = h + b2_ref[...]
    mu = jnp.mean(h, axis=-1, keepdims=True)
    d = h - mu
    var = jnp.mean(d * d, axis=-1, keepdims=True)
    o_ref[...] = x + ls_ref[...] * d * lax.rsqrt(var + 1e-5) + lb_ref[...]


def _row_spec(B, D):
    return pl.BlockSpec((B, D), lambda i: (i, 0))


def _full_spec(R, C):
    return pl.BlockSpec((R, C), lambda i: (0, 0))


@functools.cache
def _make_edge_mlp(E, D, BE):
    return pl.pallas_call(
        functools.partial(_edge_mlp_body, D),
        grid=(E // BE,),
        in_specs=[
            _row_spec(BE, D), _row_spec(BE, D), _row_spec(BE, D),
            _full_spec(3 * D, D), _full_spec(1, D),
            _full_spec(D, D), _full_spec(1, D),
            _full_spec(1, D), _full_spec(1, D),
        ],
        out_specs=_row_spec(BE, D),
        out_shape=jax.ShapeDtypeStruct((E, D), jnp.float32))


@functools.cache
def _make_node_mlp(N, D, BN):
    return pl.pallas_call(
        functools.partial(_node_mlp_body, D),
        grid=(N // BN,),
        in_specs=[
            _row_spec(BN, D), _row_spec(BN, D), _row_spec(BN, D),
            _full_spec(2 * D, D), _full_spec(1, D),
            _full_spec(D, D), _full_spec(1, D),
            _full_spec(1, D), _full_spec(1, D),
        ],
        out_specs=_row_spec(BN, D),
        out_shape=jax.ShapeDtypeStruct((N, D), jnp.float32))


def kernel(embedded_mesh_features, embedded_mesh2mesh_edge_features,
           mesh2mesh_edge_indices_src, mesh2mesh_edge_indices_dst,
           edge_w1, edge_b1, edge_w2, edge_b2, edge_ln_scale, edge_ln_bias,
           node_w1, node_b1, node_w2, node_b2, node_ln_scale, node_ln_bias):
    n_feats = embedded_mesh_features
    e_feats = embedded_mesh2mesh_edge_features
    src = mesh2mesh_edge_indices_src
    dst = mesh2mesh_edge_indices_dst

    N, D = n_feats.shape
    E = e_feats.shape[0]
    L = edge_w1.shape[0]

    gather = _make_gather(N, E, D)
    scatter = _make_scatter(N, E, D)
    edge_mlp = _make_edge_mlp(E, D, 1600)
    node_mlp = _make_node_mlp(N, D, 2000)
    zeros = jnp.zeros((N, D), jnp.float32)

    def v(x):
        return x.reshape(1, D)

    for i in range(L):
        n_src, n_dst = gather(n_feats, src, dst)
        e_feats = edge_mlp(e_feats, n_src, n_dst,
                           edge_w1[i], v(edge_b1[i]), edge_w2[i],
                           v(edge_b2[i]), v(edge_ln_scale[i]),
                           v(edge_ln_bias[i]))
        agg = scatter(e_feats, dst, zeros)
        n_feats = node_mlp(n_feats, agg[0], agg[1],
                           node_w1[i], v(node_b1[i]), node_w2[i],
                           v(node_b2[i]), v(node_ln_scale[i]),
                           v(node_ln_bias[i]))
    return (n_feats, e_feats)


# speedup vs baseline: 4.2357x; 1.5657x over previous
import functools

import jax
import jax.numpy as jnp
from jax import lax
from jax.experimental import pallas as pl
from jax.experimental.pallas import tpu as pltpu
from jax.experimental.pallas import tpu_sc as plsc

_NC = 2
_NS = 16
_NW = _NC * _NS


def _chunk_size(per_worker):
    for ch in range(128, 0, -8):
        if per_worker % ch == 0:
            return ch
    raise ValueError(per_worker)


def _sc_mesh():
    return plsc.VectorSubcoreMesh(
        core_axis_name="c", subcore_axis_name="s",
        num_cores=_NC, num_subcores=_NS)


def _gather_sc_body(EW, CH, NCH, NBUF, LOOK,
                    p_hbm, q_hbm, src_hbm, dst_hbm, out_s, out_d,
                    idxs_all, idxd_all, rows_s, rows_d,
                    gsem_s, gsem_d, wsem_s, wsem_d):
    wid = lax.axis_index("s") * _NC + lax.axis_index("c")
    base = wid * EW

    pltpu.sync_copy(src_hbm.at[pl.ds(base, EW)], idxs_all)
    pltpu.sync_copy(dst_hbm.at[pl.ds(base, EW)], idxd_all)

    def idx_s(j):
        return idxs_all.at[pl.ds(j * CH, CH)]

    def idx_d(j):
        return idxd_all.at[pl.ds(j * CH, CH)]

    def step(j, carry):
        b = lax.rem(j, NBUF)

        @pl.when(jnp.logical_and(j >= NBUF, j < NCH + NBUF))
        def _():
            jd = j - NBUF
            pltpu.make_async_copy(
                rows_s.at[b], out_s.at[pl.ds(base + jd * CH, CH)],
                wsem_s).wait()
            pltpu.make_async_copy(
                rows_d.at[b], out_d.at[pl.ds(base + jd * CH, CH)],
                wsem_d).wait()

        @pl.when(j < NCH)
        def _():
            pltpu.async_copy(p_hbm.at[idx_s(j)], rows_s.at[b], gsem_s)
            pltpu.async_copy(q_hbm.at[idx_d(j)], rows_d.at[b], gsem_d)

        @pl.when(jnp.logical_and(j >= LOOK, j < NCH + LOOK))
        def _():
            jw = j - LOOK
            bw = lax.rem(jw, NBUF)
            pltpu.make_async_copy(
                p_hbm.at[idx_s(jw)], rows_s.at[bw], gsem_s).wait()
            pltpu.make_async_copy(
                q_hbm.at[idx_d(jw)], rows_d.at[bw], gsem_d).wait()
            pltpu.async_copy(
                rows_s.at[bw], out_s.at[pl.ds(base + jw * CH, CH)], wsem_s)
            pltpu.async_copy(
                rows_d.at[bw], out_d.at[pl.ds(base + jw * CH, CH)], wsem_d)

        return carry

    lax.fori_loop(0, NCH + NBUF, step, 0)


def _scatter_sc_body(EW, CH, NCH, NBUF, LOOK,
                     e_hbm, dst_hbm, zero_hbm, agg_hbm,
                     idx_all, rows, acc, lsem, ssem):
    cid = lax.axis_index("c")
    sid = lax.axis_index("s")
    wid = sid * _NC + cid
    base = wid * EW

    @pl.when(sid == 0)
    def _():
        pltpu.sync_copy(zero_hbm, acc)

    pltpu.sync_copy(dst_hbm.at[pl.ds(base, EW)], idx_all)

    def idx(j):
        return idx_all.at[pl.ds(j * CH, CH)]

    plsc.subcore_barrier()

    def step(j, carry):
        b = lax.rem(j, NBUF)

        @pl.when(jnp.logical_and(j >= NBUF, j < NCH + NBUF))
        def _():
            jd = j - NBUF
            pltpu.make_async_copy(
                rows.at[b], acc.at[idx(jd)], ssem).wait()

        @pl.when(j < NCH)
        def _():
            pltpu.async_copy(
                e_hbm.at[pl.ds(base + j * CH, CH)], rows.at[b], lsem)

        @pl.when(jnp.logical_and(j >= LOOK, j < NCH + LOOK))
        def _():
            jw = j - LOOK
            bw = lax.rem(jw, NBUF)
            pltpu.make_async_copy(
                e_hbm.at[pl.ds(base + jw * CH, CH)], rows.at[bw],
                lsem).wait()
            pltpu.async_copy(
                rows.at[bw], acc.at[idx(jw)], ssem, add=True)

        return carry

    lax.fori_loop(0, NCH + NBUF, step, 0)
    plsc.subcore_barrier()

    @pl.when(sid == 0)
    def _():
        pltpu.sync_copy(acc, agg_hbm.at[cid])


@functools.cache
def _make_gather(N, E, D):
    EW = E // _NW
    CH = _chunk_size(EW)
    NCH = EW // CH
    NBUF, LOOK = 4, 2
    return pl.kernel(
        functools.partial(_gather_sc_body, EW, CH, NCH, NBUF, LOOK),
        out_type=(jax.ShapeDtypeStruct((E, D), jnp.float32),
                  jax.ShapeDtypeStruct((E, D), jnp.float32)),
        mesh=_sc_mesh(),
        scratch_types=[
            pltpu.VMEM((EW,), jnp.int32),
            pltpu.VMEM((EW,), jnp.int32),
            pltpu.VMEM((NBUF, CH, D), jnp.float32),
            pltpu.VMEM((NBUF, CH, D), jnp.float32),
            pltpu.SemaphoreType.DMA,
            pltpu.SemaphoreType.DMA,
            pltpu.SemaphoreType.DMA,
            pltpu.SemaphoreType.DMA,
        ])


@functools.cache
def _make_scatter(N, E, D):
    EW = E // _NW
    CH = _chunk_size(EW)
    NCH = EW // CH
    NBUF, LOOK = 4, 2
    return pl.kernel(
        functools.partial(_scatter_sc_body, EW, CH, NCH, NBUF, LOOK),
        out_type=jax.ShapeDtypeStruct((_NC, N, D), jnp.float32),
        mesh=_sc_mesh(),
        scratch_types=[
            pltpu.VMEM((EW,), jnp.int32),
            pltpu.VMEM((NBUF, CH, D), jnp.float32),
            pltpu.VMEM_SHARED((N, D), jnp.float32),
            pltpu.SemaphoreType.DMA,
            pltpu.SemaphoreType.DMA,
        ])


def _pre_body(D, n_ref, w1s_ref, w1d_ref, p_ref, q_ref):
    x = n_ref[...]
    p_ref[...] = jnp.dot(x, w1s_ref[...], preferred_element_type=jnp.float32)
    q_ref[...] = jnp.dot(x, w1d_ref[...], preferred_element_type=jnp.float32)


def _edge_mlp_body(D, e_ref, ps_ref, qd_ref, w1e_ref, b1_ref, w2_ref, b2_ref,
                   ls_ref, lb_ref, o_ref):
    x = e_ref[...]
    h = jnp.dot(x, w1e_ref[...], preferred_element_type=jnp.float32)
    h = h + ps_ref[...] + qd_ref[...] + b1_ref[...]
    h = h * jax.nn.sigmoid(h)
    h = jnp.dot(h, w2_ref[...], preferred_element_type=jnp.float32)
    h = h + b2_ref[...]
    mu = jnp.mean(h, axis=-1, keepdims=True)
    d = h - mu
    var = jnp.mean(d * d, axis=-1, keepdims=True)
    o_ref[...] = x + ls_ref[...] * d * lax.rsqrt(var + 1e-5) + lb_ref[...]


def _node_mlp_body(D, n_ref, a0_ref, a1_ref, w1_ref, b1_ref, w2_ref, b2_ref,
                   ls_ref, lb_ref, o_ref):
    x = n_ref[...]
    a = a0_ref[...] + a1_ref[...]
    h = jnp.dot(x, w1_ref[0:D], preferred_element_type=jnp.float32)
    h = h + jnp.dot(a, w1_ref[D:2 * D], preferred_element_type=jnp.float32)
    h = h + b1_ref[...]
    h = h * jax.nn.sigmoid(h)
    h = jnp.dot(h, w2_ref[...], preferred_element_type=jnp.float32)
    h = h + b2_ref[...]
    mu = jnp.mean(h, axis=-1, keepdims=True)
    d = h - mu
    var = jnp.mean(d * d, axis=-1, keepdims=True)
    o_ref[...] = x + ls_ref[...] * d * lax.rsqrt(var + 1e-5) + lb_ref[...]


def _row_spec(B, D):
    return pl.BlockSpec((B, D), lambda i: (i, 0))


def _full_spec(R, C):
    return pl.BlockSpec((R, C), lambda i: (0, 0))


@functools.cache
def _make_pre(N, D, BN):
    return pl.pallas_call(
        functools.partial(_pre_body, D),
        grid=(N // BN,),
        in_specs=[_row_spec(BN, D), _full_spec(D, D), _full_spec(D, D)],
        out_specs=(_row_spec(BN, D), _row_spec(BN, D)),
        out_shape=(jax.ShapeDtypeStruct((N, D), jnp.float32),
                   jax.ShapeDtypeStruct((N, D), jnp.float32)))


@functools.cache
def _make_edge_mlp(E, D, BE):
    return pl.pallas_call(
        functools.partial(_edge_mlp_body, D),
        grid=(E // BE,),
        in_specs=[
            _row_spec(BE, D), _row_spec(BE, D), _row_spec(BE, D),
            _full_spec(D, D), _full_spec(1, D),
            _full_spec(D, D), _full_spec(1, D),
            _full_spec(1, D), _full_spec(1, D),
        ],
        out_specs=_row_spec(BE, D),
        out_shape=jax.ShapeDtypeStruct((E, D), jnp.float32))


@functools.cache
def _make_node_mlp(N, D, BN):
    return pl.pallas_call(
        functools.partial(_node_mlp_body, D),
        grid=(N // BN,),
        in_specs=[
            _row_spec(BN, D), _row_spec(BN, D), _row_spec(BN, D),
            _full_spec(2 * D, D), _full_spec(1, D),
            _full_spec(D, D), _full_spec(1, D),
            _full_spec(1, D), _full_spec(1, D),
        ],
        out_specs=_row_spec(BN, D),
        out_shape=jax.ShapeDtypeStruct((N, D), jnp.float32))


def kernel(embedded_mesh_features, embedded_mesh2mesh_edge_features,
           mesh2mesh_edge_indices_src, mesh2mesh_edge_indices_dst,
           edge_w1, edge_b1, edge_w2, edge_b2, edge_ln_scale, edge_ln_bias,
           node_w1, node_b1, node_w2, node_b2, node_ln_scale, node_ln_bias):
    n_feats = embedded_mesh_features
    e_feats = embedded_mesh2mesh_edge_features
    src = mesh2mesh_edge_indices_src
    dst = mesh2mesh_edge_indices_dst

    N, D = n_feats.shape
    E = e_feats.shape[0]
    L = edge_w1.shape[0]

    pre = _make_pre(N, D, 2000)
    gather = _make_gather(N, E, D)
    scatter = _make_scatter(N, E, D)
    edge_mlp = _make_edge_mlp(E, D, 1600)
    node_mlp = _make_node_mlp(N, D, 2000)
    zeros = jnp.zeros((N, D), jnp.float32)

    def v(x):
        return x.reshape(1, D)

    for i in range(L):
        p, q = pre(n_feats, edge_w1[i, D:2 * D], edge_w1[i, 2 * D:3 * D])
        ps, qd = gather(p, q, src, dst)
        e_feats = edge_mlp(e_feats, ps, qd,
                           edge_w1[i, 0:D], v(edge_b1[i]), edge_w2[i],
                           v(edge_b2[i]), v(edge_ln_scale[i]),
                           v(edge_ln_bias[i]))
        agg = scatter(e_feats, dst, zeros)
        n_feats = node_mlp(n_feats, agg[0], agg[1],
                           node_w1[i], v(node_b1[i]), node_w2[i],
                           v(node_b2[i]), v(node_ln_scale[i]),
                           v(node_ln_bias[i]))
    return (n_feats, e_feats)
